# baseline (speedup 1.0000x reference)
import jax
import jax.numpy as jnp
from jax.experimental import pallas as pl

_N = 3276800
_LANES = 1024
_ROWS = (_N * 2) // _LANES
_BLOCK_R = 256


def _ew_kernel(x_ref, c1_ref, c2_ref, c3_ref, o_ref):
    x = x_ref[...]
    xl = jnp.roll(x, -1, axis=1)
    xr = jnp.roll(x, 1, axis=1)
    lane = jax.lax.broadcasted_iota(jnp.int32, x.shape, 1)
    even = (lane % 2) == 0
    sh = jnp.where(even, xl, xr)
    o_ref[...] = x * c1_ref[...] + sh * c2_ref[...] + c3_ref[...]


def kernel(keys, W, b):
    lane = jnp.arange(_LANES, dtype=jnp.int32) % 2 == 0
    c1 = jnp.where(lane, W[0, 0], W[1, 1]).reshape(1, _LANES)
    c2 = jnp.where(lane, W[0, 1], W[1, 0]).reshape(1, _LANES)
    c3 = jnp.where(lane, b[0], b[1]).reshape(1, _LANES)

    x = keys.reshape(_ROWS, _LANES)
    out = pl.pallas_call(
        _ew_kernel,
        grid=(_ROWS // _BLOCK_R,),
        in_specs=[
            pl.BlockSpec((_BLOCK_R, _LANES), lambda i: (i, 0)),
            pl.BlockSpec((1, _LANES), lambda i: (0, 0)),
            pl.BlockSpec((1, _LANES), lambda i: (0, 0)),
            pl.BlockSpec((1, _LANES), lambda i: (0, 0)),
        ],
        out_specs=pl.BlockSpec((_BLOCK_R, _LANES), lambda i: (i, 0)),
        out_shape=jax.ShapeDtypeStruct((_ROWS, _LANES), jnp.float32),
    )(x, c1, c2, c3)
    return out.reshape(_N, 2)


# speedup vs baseline: 1.0016x; 1.0016x over previous
import jax
import jax.numpy as jnp
from jax.experimental import pallas as pl

_N = 3276800
_LANES = 128
_ROWS = (_N * 2) // _LANES
_BLOCK_R = 2048


def _ew_kernel(x_ref, c1_ref, c2_ref, c3_ref, o_ref):
    x = x_ref[...]
    xl = jnp.roll(x, -1, axis=1)
    xr = jnp.roll(x, 1, axis=1)
    lane = jax.lax.broadcasted_iota(jnp.int32, x.shape, 1)
    even = (lane % 2) == 0
    sh = jnp.where(even, xl, xr)
    o_ref[...] = x * c1_ref[...] + sh * c2_ref[...] + c3_ref[...]


def kernel(keys, W, b):
    lane = jnp.arange(_LANES, dtype=jnp.int32) % 2 == 0
    c1 = jnp.where(lane, W[0, 0], W[1, 1]).reshape(1, _LANES)
    c2 = jnp.where(lane, W[0, 1], W[1, 0]).reshape(1, _LANES)
    c3 = jnp.where(lane, b[0], b[1]).reshape(1, _LANES)

    x = keys.reshape(_ROWS, _LANES)
    out = pl.pallas_call(
        _ew_kernel,
        grid=(_ROWS // _BLOCK_R,),
        in_specs=[
            pl.BlockSpec((_BLOCK_R, _LANES), lambda i: (i, 0)),
            pl.BlockSpec((1, _LANES), lambda i: (0, 0)),
            pl.BlockSpec((1, _LANES), lambda i: (0, 0)),
            pl.BlockSpec((1, _LANES), lambda i: (0, 0)),
        ],
        out_specs=pl.BlockSpec((_BLOCK_R, _LANES), lambda i: (i, 0)),
        out_shape=jax.ShapeDtypeStruct((_ROWS, _LANES), jnp.float32),
    )(x, c1, c2, c3)
    return out.reshape(_N, 2)


# speedup vs baseline: 176.6821x; 176.4044x over previous
import jax
import jax.numpy as jnp
from jax.experimental import pallas as pl

_N = 3276800
_LANES = 128
_ROWS = (_N * 2) // _LANES
_BLOCK_R = 2048


def _ew_kernel(x_ref, w_ref, o_ref):
    x = x_ref[...]
    w00 = w_ref[0, 0]
    w01 = w_ref[0, 1]
    w10 = w_ref[0, 2]
    w11 = w_ref[0, 3]
    b0 = w_ref[0, 4]
    b1 = w_ref[0, 5]
    xd = jnp.roll(x, -1, axis=0)
    xu = jnp.roll(x, 1, axis=0)
    row = jax.lax.broadcasted_iota(jnp.int32, x.shape, 0)
    even = (row % 2) == 0
    sh = jnp.where(even, xd, xu)
    c1 = jnp.where(even, w00, w11)
    c2 = jnp.where(even, w01, w10)
    c3 = jnp.where(even, b0, b1)
    o_ref[...] = x * c1 + sh * c2 + c3


def kernel(keys, W, b):
    x = keys.reshape(25600, 128, 2).transpose(0, 2, 1).reshape(_ROWS, _LANES)
    wb = jnp.concatenate(
        [W.reshape(4), b.reshape(2), jnp.zeros((122,), jnp.float32)]
    ).reshape(1, 128)
    out = pl.pallas_call(
        _ew_kernel,
        grid=(_ROWS // _BLOCK_R,),
        in_specs=[
            pl.BlockSpec((_BLOCK_R, _LANES), lambda i: (i, 0)),
            pl.BlockSpec((1, 128), lambda i: (0, 0)),
        ],
        out_specs=pl.BlockSpec((_BLOCK_R, _LANES), lambda i: (i, 0)),
        out_shape=jax.ShapeDtypeStruct((_ROWS, _LANES), jnp.float32),
    )(x, wb)
    return out.reshape(25600, 2, 128).transpose(0, 2, 1).reshape(_N, 2)
